# baseline (speedup 1.0000x reference)
import functools

import jax
import jax.numpy as jnp
from jax import lax
from jax.experimental import pallas as pl
from jax.experimental.pallas import tpu as pltpu
from jax.experimental.pallas import tpu_sc as plsc

E = 8
K = 2
MB = 128
G = 40
NW = 32
_NEG_INF = -3.0e38


def _gating_body(x_ref, Wg_ref, bg_ref, Wn_ref, bn_ref, eps_ref,
                 e0_ref, e1_ref, w0x_ref, w1x_ref, r0_ref, r1_ref,
                 imp_ref, loss_ref, off_ref, bexp_ref, bact_ref,
                 cnt_ref, accimp_ref, *, nblk):
    b = pl.program_id(0)

    @pl.when(b == 0)
    def _():
        cnt_ref[...] = jnp.zeros_like(cnt_ref)
        accimp_ref[...] = jnp.zeros_like(accimp_ref)

    @pl.when(b < nblk)
    def _():
        x = x_ref[...]
        g = jnp.dot(x, Wg_ref[...], preferred_element_type=jnp.float32) + bg_ref[...]
        z = jnp.dot(x, Wn_ref[...], preferred_element_type=jnp.float32) + bn_ref[...]
        sp = jnp.maximum(z, 0.0) + jnp.log1p(jnp.exp(-jnp.abs(z)))
        gate = g + eps_ref[...] * sp

        bn_tok = gate.shape[0]
        lane = jax.lax.broadcasted_iota(jnp.int32, (bn_tok, E), 1)
        v0 = jnp.max(gate, axis=1)
        e0 = jnp.min(jnp.where(gate == v0[:, None], lane, E), axis=1)
        masked = jnp.where(lane == e0[:, None], _NEG_INF, gate)
        v1 = jnp.max(masked, axis=1)
        e1 = jnp.min(jnp.where(masked == v1[:, None], lane, E), axis=1)

        t = jnp.exp(v1 - v0)
        w0 = 1.0 / (1.0 + t)
        w1 = t / (1.0 + t)

        e0_ref[...] = e0
        e1_ref[...] = e1
        w0x_ref[...] = jnp.broadcast_to(w0[:, None], (bn_tok, 16))
        w1x_ref[...] = jnp.broadcast_to(w1[:, None], (bn_tok, 16))

        sel0 = lane == e0[:, None]
        sel1 = lane == e1[:, None]
        imp = (jnp.where(sel0, w0[:, None], 0.0)
               + jnp.where(sel1, w1[:, None], 0.0))
        imp_ref[...] = imp
        accimp_ref[...] += jnp.sum(imp, axis=0, keepdims=True)

        oh = (sel0 | sel1).astype(jnp.float32)
        ri = jax.lax.broadcasted_iota(jnp.int32, (bn_tok, bn_tok), 0)
        ci = jax.lax.broadcasted_iota(jnp.int32, (bn_tok, bn_tok), 1)
        tri = (ri > ci).astype(jnp.float32)
        cex = jnp.dot(tri, oh, preferred_element_type=jnp.float32)
        base = cnt_ref[...]
        tot = cex + base
        r0 = jnp.sum(jnp.where(sel0, tot, 0.0), axis=1)
        r1 = jnp.sum(jnp.where(sel1, tot, 0.0), axis=1)
        r0_ref[...] = (r0 + 0.5).astype(jnp.int32)
        r1_ref[...] = (r1 + 0.5).astype(jnp.int32)
        cnt_ref[...] = base + jnp.sum(oh, axis=0, keepdims=True)

    @pl.when(b == nblk)
    def _():
        imps = accimp_ref[...]
        mean = jnp.sum(imps) / E
        var = jnp.sum((imps - mean) ** 2) / (E - 1)
        loss_ref[...] = jnp.full((1, 1), var / (mean * mean), jnp.float32)

        c = cnt_ref[...]
        ci_ = (c + 0.5).astype(jnp.int32)
        padded = ((ci_ + MB - 1) // MB) * MB
        r8 = jax.lax.broadcasted_iota(jnp.int32, (E, E), 0)
        c8 = jax.lax.broadcasted_iota(jnp.int32, (E, E), 1)
        upper = (r8 < c8).astype(jnp.float32)
        off = jnp.dot(padded.astype(jnp.float32), upper,
                      preferred_element_type=jnp.float32)
        off_i = (off + 0.5).astype(jnp.int32)
        off_ref[...] = jnp.concatenate(
            [off_i, jnp.zeros((1, 16 - E), jnp.int32)], axis=1)

        blkoff = off_i // MB
        nbl = padded // MB
        total = jnp.sum(nbl)
        lane8 = jax.lax.broadcasted_iota(jnp.int32, (1, E), 1)
        jj = jax.lax.broadcasted_iota(jnp.int32, (1, G), 1)
        acc = jnp.zeros((1, G), jnp.int32)
        for e in range(E):
            boe = jnp.sum(jnp.where(lane8 == e, blkoff, 0))
            acc = acc + (jj >= boe).astype(jnp.int32)
        bexp_ref[...] = acc - 1
        bact_ref[...] = (jj < total).astype(jnp.int32)


def _gating(x, Wg, bg, Wn, bn, eps):
    N, D = x.shape
    BN = 512
    nblk = N // BN
    cl = lambda b: (jnp.minimum(b, nblk - 1),)
    cl2 = lambda b: (jnp.minimum(b, nblk - 1), 0)
    return pl.pallas_call(
        functools.partial(_gating_body, nblk=nblk),
        grid=(nblk + 1,),
        in_specs=[
            pl.BlockSpec((BN, D), cl2),
            pl.BlockSpec((D, E), lambda b: (0, 0)),
            pl.BlockSpec((1, E), lambda b: (0, 0)),
            pl.BlockSpec((D, E), lambda b: (0, 0)),
            pl.BlockSpec((1, E), lambda b: (0, 0)),
            pl.BlockSpec((BN, E), cl2),
        ],
        out_specs=[
            pl.BlockSpec((BN,), cl),
            pl.BlockSpec((BN,), cl),
            pl.BlockSpec((BN, 16), cl2),
            pl.BlockSpec((BN, 16), cl2),
            pl.BlockSpec((BN,), cl),
            pl.BlockSpec((BN,), cl),
            pl.BlockSpec((BN, E), cl2),
            pl.BlockSpec((1, 1), lambda b: (0, 0)),
            pl.BlockSpec((1, 16), lambda b: (0, 0)),
            pl.BlockSpec((1, G), lambda b: (0, 0)),
            pl.BlockSpec((1, G), lambda b: (0, 0)),
        ],
        out_shape=[
            jax.ShapeDtypeStruct((N,), jnp.int32),
            jax.ShapeDtypeStruct((N,), jnp.int32),
            jax.ShapeDtypeStruct((N, 16), jnp.float32),
            jax.ShapeDtypeStruct((N, 16), jnp.float32),
            jax.ShapeDtypeStruct((N,), jnp.int32),
            jax.ShapeDtypeStruct((N,), jnp.int32),
            jax.ShapeDtypeStruct((N, E), jnp.float32),
            jax.ShapeDtypeStruct((1, 1), jnp.float32),
            jax.ShapeDtypeStruct((1, 16), jnp.int32),
            jax.ShapeDtypeStruct((1, G), jnp.int32),
            jax.ShapeDtypeStruct((1, G), jnp.int32),
        ],
        scratch_shapes=[pltpu.VMEM((1, E), jnp.float32),
                        pltpu.VMEM((1, E), jnp.float32)],
    )(x, Wg, bg.reshape(1, E), Wn, bn.reshape(1, E), eps)


def _dispatch_body(x_hbm, e0_hbm, e1_hbm, r0_hbm, r1_hbm, off_hbm,
                   xs_hbm, pos0_hbm, pos1_hbm,
                   e0_v, e1_v, r0_v, r1_v, off_v, pos0_v, pos1_v,
                   rowsa, rowsb, lsa, lsb, s0a, s0b, s1a, s1b, *, tpw, sub):
    wid = lax.axis_index("s") * 2 + lax.axis_index("c")
    base = wid * tpw
    pltpu.sync_copy(e0_hbm.at[pl.ds(base, tpw)], e0_v)
    pltpu.sync_copy(e1_hbm.at[pl.ds(base, tpw)], e1_v)
    pltpu.sync_copy(r0_hbm.at[pl.ds(base, tpw)], r0_v)
    pltpu.sync_copy(r1_hbm.at[pl.ds(base, tpw)], r1_v)
    pltpu.sync_copy(off_hbm, off_v)

    offv = off_v[...]

    def g16(idx):
        return jax.lax.gather(
            offv, idx[:, None],
            jax.lax.GatherDimensionNumbers(
                offset_dims=(), collapsed_slice_dims=(0,),
                start_index_map=(0,)),
            (1,), mode=jax.lax.GatherScatterMode.PROMISE_IN_BOUNDS)

    for i in range(tpw // 16):
        sl = pl.ds(i * 16, 16)
        pos0_v[sl] = r0_v[sl] + g16(e0_v[sl])
        pos1_v[sl] = r1_v[sl] + g16(e1_v[sl])
    pltpu.sync_copy(pos0_v, pos0_hbm.at[pl.ds(base, tpw)])
    pltpu.sync_copy(pos1_v, pos1_hbm.at[pl.ds(base, tpw)])

    rows = (rowsa, rowsb)
    ls = (lsa, lsb)
    s0 = (s0a, s0b)
    s1 = (s1a, s1b)
    nch = tpw // sub

    def load(c):
        par = c % 2
        return pltpu.async_copy(
            x_hbm.at[pl.ds(base + c * sub, sub)], rows[par], ls[par])

    lp = {0: load(0)}
    sp = {}
    for c in range(nch):
        par = c % 2
        lp.pop(c).wait()
        i0 = pos0_v[pl.ds(c * sub, sub)]
        h0 = pltpu.async_copy(rows[par], xs_hbm.at[i0], s0[par])
        i1 = pos1_v[pl.ds(c * sub, sub)]
        h1 = pltpu.async_copy(rows[par], xs_hbm.at[i1], s1[par])
        sp[c] = (h0, h1)
        if c + 1 < nch:
            if c - 1 >= 0:
                a, b2 = sp.pop(c - 1)
                a.wait()
                b2.wait()
            lp[c + 1] = load(c + 1)
    for c in sorted(sp):
        a, b2 = sp[c]
        a.wait()
        b2.wait()


def _dispatch(x, e0, e1, r0, r1, off16):
    N, D = x.shape
    tpw = N // NW
    sub = 16
    mesh = plsc.VectorSubcoreMesh(core_axis_name="c", subcore_axis_name="s")
    f = pl.kernel(
        functools.partial(_dispatch_body, tpw=tpw, sub=sub),
        mesh=mesh,
        out_type=[
            jax.ShapeDtypeStruct((G * MB, D), jnp.float32),
            jax.ShapeDtypeStruct((N,), jnp.int32),
            jax.ShapeDtypeStruct((N,), jnp.int32),
        ],
        scratch_types=[
            pltpu.VMEM((tpw,), jnp.int32),
            pltpu.VMEM((tpw,), jnp.int32),
            pltpu.VMEM((tpw,), jnp.int32),
            pltpu.VMEM((tpw,), jnp.int32),
            pltpu.VMEM((16,), jnp.int32),
            pltpu.VMEM((tpw,), jnp.int32),
            pltpu.VMEM((tpw,), jnp.int32),
            pltpu.VMEM((sub, D), jnp.float32),
            pltpu.VMEM((sub, D), jnp.float32),
            pltpu.SemaphoreType.DMA,
            pltpu.SemaphoreType.DMA,
            pltpu.SemaphoreType.DMA,
            pltpu.SemaphoreType.DMA,
            pltpu.SemaphoreType.DMA,
            pltpu.SemaphoreType.DMA,
        ],
    )
    return f(x, e0, e1, r0, r1, off16)


def _mm_body(bexp_s, bact_s, x_ref, We_ref, be_ref, out_ref):
    j = pl.program_id(0)

    @pl.when(bact_s[j] == 1)
    def _():
        out_ref[...] = (jnp.dot(x_ref[...], We_ref[0],
                                preferred_element_type=jnp.float32)
                        + be_ref[0, 0][None, :])


def _grouped_mm(xs, We, be, bexp, bact):
    P, D = xs.shape
    grid_spec = pltpu.PrefetchScalarGridSpec(
        num_scalar_prefetch=2,
        grid=(G,),
        in_specs=[
            pl.BlockSpec((MB, D), lambda j, bexp_s, bact_s: (j, 0)),
            pl.BlockSpec((1, D, D), lambda j, bexp_s, bact_s: (bexp_s[j], 0, 0)),
            pl.BlockSpec((1, 1, D), lambda j, bexp_s, bact_s: (bexp_s[j], 0, 0)),
        ],
        out_specs=pl.BlockSpec((MB, D), lambda j, bexp_s, bact_s: (j, 0)),
    )
    return pl.pallas_call(
        _mm_body,
        grid_spec=grid_spec,
        out_shape=jax.ShapeDtypeStruct((P, D), jnp.float32),
    )(bexp, bact, xs, We, be.reshape(E, 1, D))


def _combine_body(ys_hbm, pos0_hbm, pos1_hbm, w0x_hbm, w1x_hbm, res_hbm,
                  pos0_v, pos1_v, w0x_v, w1x_v, y0_v, y1_v, out_v,
                  sem0, sem1, *, tpw, sub, d):
    wid = lax.axis_index("s") * 2 + lax.axis_index("c")
    base = wid * tpw
    pltpu.sync_copy(pos0_hbm.at[pl.ds(base, tpw)], pos0_v)
    pltpu.sync_copy(pos1_hbm.at[pl.ds(base, tpw)], pos1_v)
    pltpu.sync_copy(w0x_hbm.at[pl.ds(base * 16, tpw * 16)], w0x_v)
    pltpu.sync_copy(w1x_hbm.at[pl.ds(base * 16, tpw * 16)], w1x_v)

    def issue(s):
        i0 = pos0_v[pl.ds(s * sub, sub)]
        c0 = pltpu.async_copy(ys_hbm.at[i0], y0_v, sem0)
        i1 = pos1_v[pl.ds(s * sub, sub)]
        c1 = pltpu.async_copy(ys_hbm.at[i1], y1_v, sem1)
        return c0, c1

    nsub = tpw // sub
    pend = issue(0)
    for s in range(nsub):
        c0, c1 = pend
        c0.wait()
        c1.wait()
        for t in range(sub):
            w0s = w0x_v[pl.ds((s * sub + t) * 16, 16)]
            w1s = w1x_v[pl.ds((s * sub + t) * 16, 16)]

            def body(jv, _):
                o = jv * 16
                out_v[t, pl.ds(o, 16)] = (y0_v[t, pl.ds(o, 16)] * w0s
                                          + y1_v[t, pl.ds(o, 16)] * w1s)
                return 0

            lax.fori_loop(0, d // 16, body, 0, unroll=8)
        if s + 1 < nsub:
            pend = issue(s + 1)
        pltpu.sync_copy(out_v, res_hbm.at[pl.ds(base + s * sub, sub)])


def _combine(ys, pos0, pos1, w0x, w1x, N, D):
    tpw = N // NW
    sub = 16
    mesh = plsc.VectorSubcoreMesh(core_axis_name="c", subcore_axis_name="s")
    f = pl.kernel(
        functools.partial(_combine_body, tpw=tpw, sub=sub, d=D),
        mesh=mesh,
        out_type=jax.ShapeDtypeStruct((N, D), jnp.float32),
        scratch_types=[
            pltpu.VMEM((tpw,), jnp.int32),
            pltpu.VMEM((tpw,), jnp.int32),
            pltpu.VMEM((tpw * 16,), jnp.float32),
            pltpu.VMEM((tpw * 16,), jnp.float32),
            pltpu.VMEM((sub, D), jnp.float32),
            pltpu.VMEM((sub, D), jnp.float32),
            pltpu.VMEM((sub, D), jnp.float32),
            pltpu.SemaphoreType.DMA,
            pltpu.SemaphoreType.DMA,
        ],
    )
    return f(ys, pos0, pos1, w0x, w1x)


def kernel(x, Wg, bg, Wn, bn, We, be, eps):
    N, D = x.shape
    (e0, e1, w0x, w1x, r0, r1, imp_full, loss, off, bexp, bact) = _gating(
        x, Wg, bg, Wn, bn, eps)
    xs, pos0, pos1 = _dispatch(x, e0, e1, r0, r1, off.reshape(16))
    ys = _grouped_mm(xs, We, be, bexp.reshape(G), bact.reshape(G))
    res = _combine(ys, pos0, pos1, w0x.reshape(N * 16), w1x.reshape(N * 16), N, D)
    return res, loss.reshape(()), imp_full


# speedup vs baseline: 1.0077x; 1.0077x over previous
import functools

import jax
import jax.numpy as jnp
from jax import lax
from jax.experimental import pallas as pl
from jax.experimental.pallas import tpu as pltpu
from jax.experimental.pallas import tpu_sc as plsc

E = 8
K = 2
MB = 128
G = 40
NW = 32
_NEG_INF = -3.0e38


def _gating_body(x_ref, Wg_ref, bg_ref, Wn_ref, bn_ref, eps_ref,
                 e0_ref, e1_ref, w0x_ref, w1x_ref, r0_ref, r1_ref,
                 imp_ref, loss_ref, off_ref, bexp_ref, bact_ref,
                 cnt_ref, accimp_ref, *, nblk):
    b = pl.program_id(0)

    @pl.when(b == 0)
    def _():
        cnt_ref[...] = jnp.zeros_like(cnt_ref)
        accimp_ref[...] = jnp.zeros_like(accimp_ref)

    @pl.when(b < nblk)
    def _():
        x = x_ref[...]
        g = jnp.dot(x, Wg_ref[...], preferred_element_type=jnp.float32) + bg_ref[...]
        z = jnp.dot(x, Wn_ref[...], preferred_element_type=jnp.float32) + bn_ref[...]
        sp = jnp.maximum(z, 0.0) + jnp.log1p(jnp.exp(-jnp.abs(z)))
        gate = g + eps_ref[...] * sp

        bn_tok = gate.shape[0]
        lane = jax.lax.broadcasted_iota(jnp.int32, (bn_tok, E), 1)
        v0 = jnp.max(gate, axis=1)
        e0 = jnp.min(jnp.where(gate == v0[:, None], lane, E), axis=1)
        masked = jnp.where(lane == e0[:, None], _NEG_INF, gate)
        v1 = jnp.max(masked, axis=1)
        e1 = jnp.min(jnp.where(masked == v1[:, None], lane, E), axis=1)

        t = jnp.exp(v1 - v0)
        w0 = 1.0 / (1.0 + t)
        w1 = t / (1.0 + t)

        e0_ref[...] = e0
        e1_ref[...] = e1
        w0x_ref[...] = jnp.broadcast_to(w0[:, None], (bn_tok, 16))
        w1x_ref[...] = jnp.broadcast_to(w1[:, None], (bn_tok, 16))

        sel0 = lane == e0[:, None]
        sel1 = lane == e1[:, None]
        imp = (jnp.where(sel0, w0[:, None], 0.0)
               + jnp.where(sel1, w1[:, None], 0.0))
        imp_ref[...] = imp
        accimp_ref[...] += jnp.sum(imp, axis=0, keepdims=True)

        oh = (sel0 | sel1).astype(jnp.float32)
        ri = jax.lax.broadcasted_iota(jnp.int32, (bn_tok, bn_tok), 0)
        ci = jax.lax.broadcasted_iota(jnp.int32, (bn_tok, bn_tok), 1)
        tri = (ri > ci).astype(jnp.float32)
        cex = jnp.dot(tri, oh, preferred_element_type=jnp.float32)
        base = cnt_ref[...]
        tot = cex + base
        r0 = jnp.sum(jnp.where(sel0, tot, 0.0), axis=1)
        r1 = jnp.sum(jnp.where(sel1, tot, 0.0), axis=1)
        r0_ref[...] = (r0 + 0.5).astype(jnp.int32)
        r1_ref[...] = (r1 + 0.5).astype(jnp.int32)
        cnt_ref[...] = base + jnp.sum(oh, axis=0, keepdims=True)

    @pl.when(b == nblk)
    def _():
        imps = accimp_ref[...]
        mean = jnp.sum(imps) / E
        var = jnp.sum((imps - mean) ** 2) / (E - 1)
        loss_ref[...] = jnp.full((1, 1), var / (mean * mean), jnp.float32)

        c = cnt_ref[...]
        ci_ = (c + 0.5).astype(jnp.int32)
        padded = ((ci_ + MB - 1) // MB) * MB
        r8 = jax.lax.broadcasted_iota(jnp.int32, (E, E), 0)
        c8 = jax.lax.broadcasted_iota(jnp.int32, (E, E), 1)
        upper = (r8 < c8).astype(jnp.float32)
        off = jnp.dot(padded.astype(jnp.float32), upper,
                      preferred_element_type=jnp.float32)
        off_i = (off + 0.5).astype(jnp.int32)
        off_ref[...] = jnp.concatenate(
            [off_i, jnp.zeros((1, 16 - E), jnp.int32)], axis=1)

        blkoff = off_i // MB
        nbl = padded // MB
        total = jnp.sum(nbl)
        lane8 = jax.lax.broadcasted_iota(jnp.int32, (1, E), 1)
        jj = jax.lax.broadcasted_iota(jnp.int32, (1, G), 1)
        acc = jnp.zeros((1, G), jnp.int32)
        for e in range(E):
            boe = jnp.sum(jnp.where(lane8 == e, blkoff, 0))
            acc = acc + (jj >= boe).astype(jnp.int32)
        bexp_ref[...] = acc - 1
        bact_ref[...] = (jj < total).astype(jnp.int32)


def _gating(x, Wg, bg, Wn, bn, eps):
    N, D = x.shape
    BN = 256
    nblk = N // BN
    cl = lambda b: (jnp.minimum(b, nblk - 1),)
    cl2 = lambda b: (jnp.minimum(b, nblk - 1), 0)
    return pl.pallas_call(
        functools.partial(_gating_body, nblk=nblk),
        grid=(nblk + 1,),
        in_specs=[
            pl.BlockSpec((BN, D), cl2),
            pl.BlockSpec((D, E), lambda b: (0, 0)),
            pl.BlockSpec((1, E), lambda b: (0, 0)),
            pl.BlockSpec((D, E), lambda b: (0, 0)),
            pl.BlockSpec((1, E), lambda b: (0, 0)),
            pl.BlockSpec((BN, E), cl2),
        ],
        out_specs=[
            pl.BlockSpec((BN,), cl),
            pl.BlockSpec((BN,), cl),
            pl.BlockSpec((BN, 16), cl2),
            pl.BlockSpec((BN, 16), cl2),
            pl.BlockSpec((BN,), cl),
            pl.BlockSpec((BN,), cl),
            pl.BlockSpec((BN, E), cl2),
            pl.BlockSpec((1, 1), lambda b: (0, 0)),
            pl.BlockSpec((1, 16), lambda b: (0, 0)),
            pl.BlockSpec((1, G), lambda b: (0, 0)),
            pl.BlockSpec((1, G), lambda b: (0, 0)),
        ],
        out_shape=[
            jax.ShapeDtypeStruct((N,), jnp.int32),
            jax.ShapeDtypeStruct((N,), jnp.int32),
            jax.ShapeDtypeStruct((N, 16), jnp.float32),
            jax.ShapeDtypeStruct((N, 16), jnp.float32),
            jax.ShapeDtypeStruct((N,), jnp.int32),
            jax.ShapeDtypeStruct((N,), jnp.int32),
            jax.ShapeDtypeStruct((N, E), jnp.float32),
            jax.ShapeDtypeStruct((1, 1), jnp.float32),
            jax.ShapeDtypeStruct((1, 16), jnp.int32),
            jax.ShapeDtypeStruct((1, G), jnp.int32),
            jax.ShapeDtypeStruct((1, G), jnp.int32),
        ],
        scratch_shapes=[pltpu.VMEM((1, E), jnp.float32),
                        pltpu.VMEM((1, E), jnp.float32)],
    )(x, Wg, bg.reshape(1, E), Wn, bn.reshape(1, E), eps)


def _dispatch_body(x_hbm, e0_hbm, e1_hbm, r0_hbm, r1_hbm, off_hbm,
                   xs_hbm, pos0_hbm, pos1_hbm,
                   e0_v, e1_v, r0_v, r1_v, off_v, pos0_v, pos1_v,
                   rowsa, rowsb, lsa, lsb, s0a, s0b, s1a, s1b, *, tpw, sub):
    wid = lax.axis_index("s") * 2 + lax.axis_index("c")
    base = wid * tpw
    pltpu.sync_copy(e0_hbm.at[pl.ds(base, tpw)], e0_v)
    pltpu.sync_copy(e1_hbm.at[pl.ds(base, tpw)], e1_v)
    pltpu.sync_copy(r0_hbm.at[pl.ds(base, tpw)], r0_v)
    pltpu.sync_copy(r1_hbm.at[pl.ds(base, tpw)], r1_v)
    pltpu.sync_copy(off_hbm, off_v)

    offv = off_v[...]

    def g16(idx):
        return jax.lax.gather(
            offv, idx[:, None],
            jax.lax.GatherDimensionNumbers(
                offset_dims=(), collapsed_slice_dims=(0,),
                start_index_map=(0,)),
            (1,), mode=jax.lax.GatherScatterMode.PROMISE_IN_BOUNDS)

    for i in range(tpw // 16):
        sl = pl.ds(i * 16, 16)
        pos0_v[sl] = r0_v[sl] + g16(e0_v[sl])
        pos1_v[sl] = r1_v[sl] + g16(e1_v[sl])
    pltpu.sync_copy(pos0_v, pos0_hbm.at[pl.ds(base, tpw)])
    pltpu.sync_copy(pos1_v, pos1_hbm.at[pl.ds(base, tpw)])

    rows = (rowsa, rowsb)
    ls = (lsa, lsb)
    s0 = (s0a, s0b)
    s1 = (s1a, s1b)
    nch = tpw // sub

    def load(c):
        par = c % 2
        return pltpu.async_copy(
            x_hbm.at[pl.ds(base + c * sub, sub)], rows[par], ls[par])

    lp = {0: load(0)}
    sp = {}
    for c in range(nch):
        par = c % 2
        lp.pop(c).wait()
        i0 = pos0_v[pl.ds(c * sub, sub)]
        h0 = pltpu.async_copy(rows[par], xs_hbm.at[i0], s0[par])
        i1 = pos1_v[pl.ds(c * sub, sub)]
        h1 = pltpu.async_copy(rows[par], xs_hbm.at[i1], s1[par])
        sp[c] = (h0, h1)
        if c + 1 < nch:
            if c - 1 >= 0:
                a, b2 = sp.pop(c - 1)
                a.wait()
                b2.wait()
            lp[c + 1] = load(c + 1)
    for c in sorted(sp):
        a, b2 = sp[c]
        a.wait()
        b2.wait()


def _dispatch(x, e0, e1, r0, r1, off16):
    N, D = x.shape
    tpw = N // NW
    sub = 16
    mesh = plsc.VectorSubcoreMesh(core_axis_name="c", subcore_axis_name="s")
    f = pl.kernel(
        functools.partial(_dispatch_body, tpw=tpw, sub=sub),
        mesh=mesh,
        out_type=[
            jax.ShapeDtypeStruct((G * MB, D), jnp.float32),
            jax.ShapeDtypeStruct((N,), jnp.int32),
            jax.ShapeDtypeStruct((N,), jnp.int32),
        ],
        scratch_types=[
            pltpu.VMEM((tpw,), jnp.int32),
            pltpu.VMEM((tpw,), jnp.int32),
            pltpu.VMEM((tpw,), jnp.int32),
            pltpu.VMEM((tpw,), jnp.int32),
            pltpu.VMEM((16,), jnp.int32),
            pltpu.VMEM((tpw,), jnp.int32),
            pltpu.VMEM((tpw,), jnp.int32),
            pltpu.VMEM((sub, D), jnp.float32),
            pltpu.VMEM((sub, D), jnp.float32),
            pltpu.SemaphoreType.DMA,
            pltpu.SemaphoreType.DMA,
            pltpu.SemaphoreType.DMA,
            pltpu.SemaphoreType.DMA,
            pltpu.SemaphoreType.DMA,
            pltpu.SemaphoreType.DMA,
        ],
    )
    return f(x, e0, e1, r0, r1, off16)


def _mm_body(bexp_s, bact_s, x_ref, We_ref, be_ref, out_ref):
    j = pl.program_id(0)

    @pl.when(bact_s[j] == 1)
    def _():
        out_ref[...] = (jnp.dot(x_ref[...], We_ref[0],
                                preferred_element_type=jnp.float32)
                        + be_ref[0, 0][None, :])


def _grouped_mm(xs, We, be, bexp, bact):
    P, D = xs.shape
    grid_spec = pltpu.PrefetchScalarGridSpec(
        num_scalar_prefetch=2,
        grid=(G,),
        in_specs=[
            pl.BlockSpec((MB, D), lambda j, bexp_s, bact_s: (j, 0)),
            pl.BlockSpec((1, D, D), lambda j, bexp_s, bact_s: (bexp_s[j], 0, 0)),
            pl.BlockSpec((1, 1, D), lambda j, bexp_s, bact_s: (bexp_s[j], 0, 0)),
        ],
        out_specs=pl.BlockSpec((MB, D), lambda j, bexp_s, bact_s: (j, 0)),
    )
    return pl.pallas_call(
        _mm_body,
        grid_spec=grid_spec,
        out_shape=jax.ShapeDtypeStruct((P, D), jnp.float32),
    )(bexp, bact, xs, We, be.reshape(E, 1, D))


def _combine_body(ys_hbm, pos0_hbm, pos1_hbm, w0x_hbm, w1x_hbm, res_hbm,
                  pos0_v, pos1_v, w0x_v, w1x_v, y0_v, y1_v, out_v,
                  sem0, sem1, *, tpw, sub, d):
    wid = lax.axis_index("s") * 2 + lax.axis_index("c")
    base = wid * tpw
    pltpu.sync_copy(pos0_hbm.at[pl.ds(base, tpw)], pos0_v)
    pltpu.sync_copy(pos1_hbm.at[pl.ds(base, tpw)], pos1_v)
    pltpu.sync_copy(w0x_hbm.at[pl.ds(base * 16, tpw * 16)], w0x_v)
    pltpu.sync_copy(w1x_hbm.at[pl.ds(base * 16, tpw * 16)], w1x_v)

    def issue(s):
        i0 = pos0_v[pl.ds(s * sub, sub)]
        c0 = pltpu.async_copy(ys_hbm.at[i0], y0_v, sem0)
        i1 = pos1_v[pl.ds(s * sub, sub)]
        c1 = pltpu.async_copy(ys_hbm.at[i1], y1_v, sem1)
        return c0, c1

    nsub = tpw // sub
    pend = issue(0)
    for s in range(nsub):
        c0, c1 = pend
        c0.wait()
        c1.wait()
        for t in range(sub):
            w0s = w0x_v[pl.ds((s * sub + t) * 16, 16)]
            w1s = w1x_v[pl.ds((s * sub + t) * 16, 16)]

            def body(jv, _):
                o = jv * 16
                out_v[t, pl.ds(o, 16)] = (y0_v[t, pl.ds(o, 16)] * w0s
                                          + y1_v[t, pl.ds(o, 16)] * w1s)
                return 0

            lax.fori_loop(0, d // 16, body, 0, unroll=8)
        if s + 1 < nsub:
            pend = issue(s + 1)
        pltpu.sync_copy(out_v, res_hbm.at[pl.ds(base + s * sub, sub)])


def _combine(ys, pos0, pos1, w0x, w1x, N, D):
    tpw = N // NW
    sub = 16
    mesh = plsc.VectorSubcoreMesh(core_axis_name="c", subcore_axis_name="s")
    f = pl.kernel(
        functools.partial(_combine_body, tpw=tpw, sub=sub, d=D),
        mesh=mesh,
        out_type=jax.ShapeDtypeStruct((N, D), jnp.float32),
        scratch_types=[
            pltpu.VMEM((tpw,), jnp.int32),
            pltpu.VMEM((tpw,), jnp.int32),
            pltpu.VMEM((tpw * 16,), jnp.float32),
            pltpu.VMEM((tpw * 16,), jnp.float32),
            pltpu.VMEM((sub, D), jnp.float32),
            pltpu.VMEM((sub, D), jnp.float32),
            pltpu.VMEM((sub, D), jnp.float32),
            pltpu.SemaphoreType.DMA,
            pltpu.SemaphoreType.DMA,
        ],
    )
    return f(ys, pos0, pos1, w0x, w1x)


def kernel(x, Wg, bg, Wn, bn, We, be, eps):
    N, D = x.shape
    (e0, e1, w0x, w1x, r0, r1, imp_full, loss, off, bexp, bact) = _gating(
        x, Wg, bg, Wn, bn, eps)
    xs, pos0, pos1 = _dispatch(x, e0, e1, r0, r1, off.reshape(16))
    ys = _grouped_mm(xs, We, be, bexp.reshape(G), bact.reshape(G))
    res = _combine(ys, pos0, pos1, w0x.reshape(N * 16), w1x.reshape(N * 16), N, D)
    return res, loss.reshape(()), imp_full
